# baseline (speedup 1.0000x reference)
import jax
import jax.numpy as jnp
from jax.experimental import pallas as pl
from jax.experimental.pallas import tpu as pltpu

_MARGIN = 0.2
_PAD_DIST = 100.0
_BIG = 1e9


def _make_body(bb, p, d):

    def body(m1_ref, m2_ref, gt_ref, e1_ref, e2_ref, out_ref, acc_ref):
        i = pl.program_id(0)

        @pl.when(i == 0)
        def _init():
            acc_ref[0] = 0.0
            acc_ref[1] = 0.0

        e1 = e1_ref[...]
        e2 = e2_ref[...]
        gtb = gt_ref[...] != 0
        onescol = jnp.ones((bb, p, 1), dtype=jnp.float32)
        a2 = jnp.sum(e1 * e1, axis=2, keepdims=True)
        b2 = jnp.sum(e2 * e2, axis=2, keepdims=True)
        u = jnp.concatenate([-2.0 * e1, a2, onescol], axis=2)
        v = jnp.concatenate([e2, onescol, b2], axis=2)
        d2 = jax.lax.dot_general(u, v, (((2,), (2,)), ((0,), (0,))),
                                 preferred_element_type=jnp.float32)
        dist = jnp.sqrt(jnp.maximum(d2, 0.0))
        validm = (m1_ref[...] != 0) & (m2_ref[...] != 0)
        df = jnp.where(validm, dist, _PAD_DIST)
        mn = jnp.min(jnp.where(gtb, _BIG, df), axis=2, keepdims=True)
        val = df + _MARGIN - mn
        sel = gtb & (val > 0.0)
        acc_ref[0] += jnp.sum(jnp.where(sel, val, 0.0))
        acc_ref[1] += jnp.sum(sel.astype(jnp.float32))

        @pl.when(i == pl.num_programs(0) - 1)
        def _fin():
            total = acc_ref[0]
            cnt = acc_ref[1]
            out_ref[0, 0] = jnp.where(cnt > 0.0,
                                      total / jnp.maximum(cnt, 1.0), _MARGIN)

    return body


def kernel(embeddings1, embeddings2, gt_corr_ms, numPlanes1, numPlanes2,
           loss_weight):
    B, P, D = embeddings1.shape
    BB = 128
    nblk = B // BB
    r = jnp.arange(P)
    m1 = (r[None, :] < numPlanes1[:, None]).astype(jnp.int8)[:, :, None]
    m2 = (r[None, :] < numPlanes2[:, None]).astype(jnp.int8)[:, None, :]
    gt8 = gt_corr_ms.view(jnp.int8)
    out = pl.pallas_call(
        _make_body(BB, P, D),
        grid=(nblk,),
        in_specs=[
            pl.BlockSpec((BB, P, 1), lambda i: (i, 0, 0)),
            pl.BlockSpec((BB, 1, P), lambda i: (i, 0, 0)),
            pl.BlockSpec((BB, P, P), lambda i: (i, 0, 0)),
            pl.BlockSpec((BB, P, D), lambda i: (i, 0, 0)),
            pl.BlockSpec((BB, P, D), lambda i: (i, 0, 0)),
        ],
        out_specs=pl.BlockSpec((1, 1), lambda i: (0, 0),
                               memory_space=pltpu.SMEM),
        out_shape=jax.ShapeDtypeStruct((1, 1), jnp.float32),
        scratch_shapes=[pltpu.SMEM((2,), jnp.float32)],
    )(m1, m2, gt8, embeddings1, embeddings2)
    return (loss_weight * out[0, 0]).astype(jnp.float32)


# speedup vs baseline: 3.3109x; 3.3109x over previous
import jax
import jax.numpy as jnp
from jax.experimental import pallas as pl
from jax.experimental.pallas import tpu as pltpu

_MARGIN = 0.2
_PAD_DIST = 100.0
_BIG = 1e9


def _make_body(bb, p, d):

    def body(code_ref, e1_ref, e2_ref, out_ref, acc_ref):
        i = pl.program_id(0)

        @pl.when(i == 0)
        def _init():
            acc_ref[0] = 0.0
            acc_ref[1] = 0.0

        e1 = e1_ref[...]
        e2 = e2_ref[...]
        code = code_ref[...]
        onescol = jnp.ones((bb, p, 1), dtype=jnp.float32)
        a2 = jnp.sum(e1 * e1, axis=2, keepdims=True)
        b2 = jnp.sum(e2 * e2, axis=2, keepdims=True)
        u = jnp.concatenate([-2.0 * e1, a2, onescol], axis=2)
        v = jnp.concatenate([e2, onescol, b2], axis=2)
        d2 = jax.lax.dot_general(u, v, (((2,), (2,)), ((0,), (0,))),
                                 preferred_element_type=jnp.float32)
        dist = jnp.sqrt(jnp.maximum(d2, 0.0))
        validm = (code == 1.0) | (code == 3.0)
        gtb = code >= 2.0
        df = jnp.where(validm, dist, _PAD_DIST)
        mn = jnp.min(jnp.where(gtb, _BIG, df), axis=2, keepdims=True)
        val = df + _MARGIN - mn
        sel = gtb & (val > 0.0)
        acc_ref[0] += jnp.sum(jnp.where(sel, val, 0.0))
        acc_ref[1] += jnp.sum(sel.astype(jnp.float32))

        @pl.when(i == pl.num_programs(0) - 1)
        def _fin():
            total = acc_ref[0]
            cnt = acc_ref[1]
            out_ref[0, 0] = jnp.where(cnt > 0.0,
                                      total / jnp.maximum(cnt, 1.0), _MARGIN)

    return body


def kernel(embeddings1, embeddings2, gt_corr_ms, numPlanes1, numPlanes2,
           loss_weight):
    B, P, D = embeddings1.shape
    BB = 128
    nblk = B // BB
    r = jnp.arange(P)
    m1 = (r[None, :] < numPlanes1[:, None]).astype(jnp.float32)
    m2 = (r[None, :] < numPlanes2[:, None]).astype(jnp.float32)
    code = (m1[:, :, None] * m2[:, None, :]
            + 2.0 * gt_corr_ms.astype(jnp.float32))
    out = pl.pallas_call(
        _make_body(BB, P, D),
        grid=(nblk,),
        in_specs=[
            pl.BlockSpec((BB, P, P), lambda i: (i, 0, 0)),
            pl.BlockSpec((BB, P, D), lambda i: (i, 0, 0)),
            pl.BlockSpec((BB, P, D), lambda i: (i, 0, 0)),
        ],
        out_specs=pl.BlockSpec((1, 1), lambda i: (0, 0),
                               memory_space=pltpu.SMEM),
        out_shape=jax.ShapeDtypeStruct((1, 1), jnp.float32),
        scratch_shapes=[pltpu.SMEM((2,), jnp.float32)],
    )(code, embeddings1, embeddings2)
    return (loss_weight * out[0, 0]).astype(jnp.float32)
